# baseline (speedup 1.0000x reference)
import jax
import jax.numpy as jnp
import numpy as np
from jax.experimental import pallas as pl

_ROWS = 1_000_000
_DIM = 64
_PAIR_ROWS = _ROWS // 2
_BLOCK_ROWS = 4_000

def _l2norm_body(x_ref, o_ref):
    x = x_ref[...]
    r = jax.lax.broadcasted_iota(jnp.int32, (128, 128), 0)
    c = jax.lax.broadcasted_iota(jnp.int32, (128, 128), 1)
    m = jnp.where((r // 64) == (c // 64), 1.0, 0.0).astype(jnp.float32)
    n = jax.lax.dot(x * x, m, preferred_element_type=jnp.float32)
    o_ref[...] = x * jax.lax.rsqrt(n)


def kernel(weight):
    w2 = weight.reshape(_PAIR_ROWS, 2 * _DIM)
    out = pl.pallas_call(
        _l2norm_body,
        grid=(_PAIR_ROWS // _BLOCK_ROWS,),
        in_specs=[pl.BlockSpec((_BLOCK_ROWS, 2 * _DIM), lambda i: (i, 0))],
        out_specs=pl.BlockSpec((_BLOCK_ROWS, 2 * _DIM), lambda i: (i, 0)),
        out_shape=jax.ShapeDtypeStruct((_PAIR_ROWS, 2 * _DIM), jnp.float32),
    )(w2)
    return out.reshape(_ROWS, _DIM)


# speedup vs baseline: 1.3916x; 1.3916x over previous
import jax
import jax.numpy as jnp
from jax.experimental import pallas as pl

_ROWS = 1_000_000
_DIM = 64
_BLOCK_ROWS = 20_000


def _l2norm_body(x_ref, o_ref):
    x = x_ref[...]
    ones = jnp.ones((_DIM, _DIM), dtype=jnp.float32)
    n = jax.lax.dot(x * x, ones, preferred_element_type=jnp.float32)
    o_ref[...] = x * jax.lax.rsqrt(n)


def kernel(weight):
    return pl.pallas_call(
        _l2norm_body,
        grid=(_ROWS // _BLOCK_ROWS,),
        in_specs=[pl.BlockSpec((_BLOCK_ROWS, _DIM), lambda i: (i, 0))],
        out_specs=pl.BlockSpec((_BLOCK_ROWS, _DIM), lambda i: (i, 0)),
        out_shape=jax.ShapeDtypeStruct((_ROWS, _DIM), jnp.float32),
    )(weight)
